# initial kernel scaffold (unmeasured)
import jax
import jax.numpy as jnp
from jax import lax
from jax.experimental import pallas as pl
from jax.experimental.pallas import tpu as pltpu

N_DEV = 4
M = 8192
M_PER = M // N_DEV
N = 4096
NQ = 4
NQ_W = N // NQ


def kernel(x, w_mat):
    x = x.astype(jnp.bfloat16)
    w = w_mat.astype(jnp.bfloat16)
    k_per = x.shape[1]

    def body(x_hbm, w_ref, out_ref, xbuf, comm_ref, load_sem, send_sems, recv_sems):
        my = lax.axis_index("i")
        left = lax.rem(my + N_DEV - 1, N_DEV)
        right = lax.rem(my + 1, N_DEV)

        barrier = pltpu.get_barrier_semaphore()
        pl.semaphore_signal(barrier, inc=1, device_id=(left,),
                            device_id_type=pl.DeviceIdType.MESH)
        pl.semaphore_signal(barrier, inc=1, device_id=(right,),
                            device_id_type=pl.DeviceIdType.MESH)
        pl.semaphore_wait(barrier, 2)

        def load_x(c):
            return pltpu.make_async_copy(
                x_hbm.at[pl.ds(c * M_PER, M_PER), :], xbuf, load_sem)

        cp = load_x(lax.rem(my + N_DEV - 1, N_DEV))
        cp.start()
        cp.wait()
        for q in range(NQ):
            sl = pl.ds(q * NQ_W, NQ_W)
            p = jnp.dot(xbuf[:, :], w_ref[:, sl],
                        preferred_element_type=jnp.float32)
            comm_ref[0, :, sl] = p.astype(jnp.bfloat16)

        for h in range(N_DEV - 1):
            send_slot = h % 2
            recv_slot = (h + 1) % 2
            rdma = pltpu.make_async_remote_copy(
                src_ref=comm_ref.at[send_slot],
                dst_ref=comm_ref.at[recv_slot],
                send_sem=send_sems.at[send_slot],
                recv_sem=recv_sems.at[recv_slot],
                device_id=(right,),
                device_id_type=pl.DeviceIdType.MESH,
            )
            rdma.start()
            cp = load_x(lax.rem(my + 2 * N_DEV - 2 - h, N_DEV))
            cp.start()
            cp.wait()
            rdma.wait()
            last = h == N_DEV - 2
            for q in range(NQ):
                sl = pl.ds(q * NQ_W, NQ_W)
                p = jnp.dot(xbuf[:, :], w_ref[:, sl],
                            preferred_element_type=jnp.float32)
                acc = comm_ref[recv_slot, :, sl].astype(jnp.float32) + p
                if last:
                    out_ref[:, sl] = acc
                else:
                    comm_ref[recv_slot, :, sl] = acc.astype(jnp.bfloat16)

    return pl.pallas_call(
        body,
        out_shape=jax.ShapeDtypeStruct((M_PER, N), jnp.float32),
        in_specs=[
            pl.BlockSpec(memory_space=pltpu.ANY),
            pl.BlockSpec(memory_space=pltpu.VMEM),
        ],
        out_specs=pl.BlockSpec(memory_space=pltpu.VMEM),
        scratch_shapes=[
            pltpu.VMEM((M_PER, k_per), jnp.bfloat16),
            pltpu.VMEM((2, M_PER, N), jnp.bfloat16),
            pltpu.SemaphoreType.DMA,
            pltpu.SemaphoreType.DMA((2,)),
            pltpu.SemaphoreType.DMA((2,)),
        ],
        compiler_params=pltpu.CompilerParams(collective_id=0),
    )(x, w)


# baseline (device time: 804208 ns/iter reference)
import jax
import jax.numpy as jnp
from jax import lax
from jax.experimental import pallas as pl
from jax.experimental.pallas import tpu as pltpu

N_DEV = 4
M = 8192
M_PER = M // N_DEV
N = 4096
N_HALF = N // 2
RS = 512
N_STRIPS = M_PER // RS


def kernel(x, w_mat):
    x = x.astype(jnp.bfloat16)
    w = w_mat.astype(jnp.bfloat16)
    k_per = x.shape[1]

    def body(x_hbm, w_hbm, out_hbm, xbuf, wbuf, comm_ref, stage,
             ld_sem, st_sem, send_sems, recv_sems):
        my = lax.axis_index("i")
        left = lax.rem(my + N_DEV - 1, N_DEV)
        right = lax.rem(my + 1, N_DEV)

        barrier = pltpu.get_barrier_semaphore()
        pl.semaphore_signal(barrier, inc=1, device_id=(left,),
                            device_id_type=pl.DeviceIdType.MESH)
        pl.semaphore_signal(barrier, inc=1, device_id=(right,),
                            device_id_type=pl.DeviceIdType.MESH)
        pl.semaphore_wait(barrier, 2)

        def load_x(c):
            cp = pltpu.make_async_copy(
                x_hbm.at[pl.ds(c * M_PER, M_PER), :], xbuf, ld_sem)
            cp.start()
            cp.wait()

        def strip_dot(r):
            return jnp.dot(xbuf[pl.ds(r * RS, RS), :], wbuf[:, :],
                           preferred_element_type=jnp.float32)

        def run_pass(p, _):
            n_off = p * N_HALF
            cp = pltpu.make_async_copy(
                w_hbm.at[:, pl.ds(n_off, N_HALF)], wbuf, ld_sem)
            cp.start()
            cp.wait()

            load_x(lax.rem(my + N_DEV - 1, N_DEV))

            def s0(r, carry):
                comm_ref[0, pl.ds(r * RS, RS), :] = (
                    strip_dot(r).astype(jnp.bfloat16))
                return carry
            lax.fori_loop(0, N_STRIPS, s0, 0)

            for h in range(N_DEV - 1):
                send_slot = h % 2
                recv_slot = (h + 1) % 2
                rdma = pltpu.make_async_remote_copy(
                    src_ref=comm_ref.at[send_slot],
                    dst_ref=comm_ref.at[recv_slot],
                    send_sem=send_sems.at[send_slot],
                    recv_sem=recv_sems.at[recv_slot],
                    device_id=(right,),
                    device_id_type=pl.DeviceIdType.MESH,
                )
                rdma.start()
                load_x(lax.rem(my + 2 * N_DEV - 2 - h, N_DEV))
                rdma.wait()

                if h < N_DEV - 2:
                    def mid(r, carry):
                        sl = pl.ds(r * RS, RS)
                        acc = (comm_ref[recv_slot, sl, :]
                               .astype(jnp.float32) + strip_dot(r))
                        comm_ref[recv_slot, sl, :] = acc.astype(jnp.bfloat16)
                        return carry
                    lax.fori_loop(0, N_STRIPS, mid, 0)
                else:
                    def last(r, carry):
                        sl = pl.ds(r * RS, RS)
                        stage[sl, :] = (comm_ref[recv_slot, sl, :]
                                        .astype(jnp.float32) + strip_dot(r))
                        return carry
                    lax.fori_loop(0, N_STRIPS, last, 0)
                    st = pltpu.make_async_copy(
                        stage, out_hbm.at[:, pl.ds(n_off, N_HALF)], st_sem)
                    st.start()
                    st.wait()
            return _

        lax.fori_loop(0, 2, run_pass, 0)

    return pl.pallas_call(
        body,
        out_shape=jax.ShapeDtypeStruct((M_PER, N), jnp.float32),
        in_specs=[
            pl.BlockSpec(memory_space=pl.ANY),
            pl.BlockSpec(memory_space=pl.ANY),
        ],
        out_specs=pl.BlockSpec(memory_space=pl.ANY),
        scratch_shapes=[
            pltpu.VMEM((M_PER, k_per), jnp.bfloat16),
            pltpu.VMEM((k_per, N_HALF), jnp.bfloat16),
            pltpu.VMEM((2, M_PER, N_HALF), jnp.bfloat16),
            pltpu.VMEM((M_PER, N_HALF), jnp.float32),
            pltpu.SemaphoreType.DMA,
            pltpu.SemaphoreType.DMA,
            pltpu.SemaphoreType.DMA((2,)),
            pltpu.SemaphoreType.DMA((2,)),
        ],
        compiler_params=pltpu.CompilerParams(
            collective_id=0,
            vmem_limit_bytes=64 * 1024 * 1024,
        ),
    )(x, w)


# device time: 428306 ns/iter; 1.8776x vs baseline; 1.8776x over previous
import jax
import jax.numpy as jnp
from jax import lax
from jax.experimental import pallas as pl
from jax.experimental.pallas import tpu as pltpu

N_DEV = 4
M = 8192
M_PER = M // N_DEV
RH = M_PER // 2
N = 4096
N_HALF = N // 2
RS = 512
N_STRIPS = RH // RS


def kernel(x, w_mat):
    x = x.astype(jnp.bfloat16)
    w = w_mat.astype(jnp.bfloat16)
    k_per = x.shape[1]

    def body(x_hbm, w_hbm, out_hbm,
             xa, xb, wbuf, comm_r, comm_l, pb_r, pb_l, stage,
             ldw_sem, lda_sem, ldb_sem, st_sem,
             send_r, recv_r, send_l, recv_l):
        my = lax.axis_index("i")
        left = lax.rem(my + N_DEV - 1, N_DEV)
        right = lax.rem(my + 1, N_DEV)

        barrier = pltpu.get_barrier_semaphore()
        pl.semaphore_signal(barrier, inc=1, device_id=(left,),
                            device_id_type=pl.DeviceIdType.MESH)
        pl.semaphore_signal(barrier, inc=1, device_id=(right,),
                            device_id_type=pl.DeviceIdType.MESH)
        pl.semaphore_wait(barrier, 2)

        def load_half(c, row_off, buf, sem):
            cp = pltpu.make_async_copy(
                x_hbm.at[pl.ds(c * M_PER + row_off, RH), :], buf, sem)
            cp.start()
            return cp

        def dots_into(dst, src):
            def step(r, carry):
                sl = pl.ds(r * RS, RS)
                dst[sl, :] = jnp.dot(
                    src[sl, :], wbuf[:, :],
                    preferred_element_type=jnp.float32).astype(jnp.bfloat16)
                return carry
            lax.fori_loop(0, N_STRIPS, step, 0)

        def acc_mid(comm, slot, pb):
            def step(r, carry):
                sl = pl.ds(r * RS, RS)
                comm[slot, sl, :] = (
                    comm[slot, sl, :].astype(jnp.float32)
                    + pb[sl, :].astype(jnp.float32)).astype(jnp.bfloat16)
                return carry
            lax.fori_loop(0, N_STRIPS, step, 0)

        def acc_last(comm, slot, pb, stage_row):
            def step(r, carry):
                sl = pl.ds(r * RS, RS)
                stage[pl.ds(stage_row + r * RS, RS), :] = (
                    comm[slot, sl, :].astype(jnp.float32)
                    + pb[sl, :].astype(jnp.float32))
                return carry
            lax.fori_loop(0, N_STRIPS, step, 0)

        def run_pass(p, _):
            n_off = p * N_HALF
            cp = pltpu.make_async_copy(
                w_hbm.at[:, pl.ds(n_off, N_HALF)], wbuf, ldw_sem)
            cp.start()
            cp.wait()

            ca = load_half(lax.rem(my + N_DEV - 1, N_DEV), 0, xa, lda_sem)
            cb = load_half(lax.rem(my + 1, N_DEV), RH, xb, ldb_sem)
            ca.wait()
            dots_into(comm_r.at[0], xa)
            cb.wait()
            dots_into(comm_l.at[0], xb)

            for h in range(N_DEV - 1):
                s = h % 2
                d = (h + 1) % 2
                rdma_r = pltpu.make_async_remote_copy(
                    src_ref=comm_r.at[s], dst_ref=comm_r.at[d],
                    send_sem=send_r.at[s], recv_sem=recv_r.at[d],
                    device_id=(right,), device_id_type=pl.DeviceIdType.MESH)
                rdma_l = pltpu.make_async_remote_copy(
                    src_ref=comm_l.at[s], dst_ref=comm_l.at[d],
                    send_sem=send_l.at[s], recv_sem=recv_l.at[d],
                    device_id=(left,), device_id_type=pl.DeviceIdType.MESH)
                rdma_r.start()
                rdma_l.start()
                ca = load_half(lax.rem(my + 2 * N_DEV - 2 - h, N_DEV),
                               0, xa, lda_sem)
                cb = load_half(lax.rem(my + 2 + h, N_DEV), RH, xb, ldb_sem)
                ca.wait()
                dots_into(pb_r, xa)
                cb.wait()
                dots_into(pb_l, xb)

                last = h == N_DEV - 2
                rdma_r.wait()
                if not last:
                    acc_mid(comm_r, d, pb_r)
                    rdma_l.wait()
                    acc_mid(comm_l, d, pb_l)
                else:
                    acc_last(comm_r, d, pb_r, 0)
                    rdma_l.wait()
                    acc_last(comm_l, d, pb_l, RH)
                    st = pltpu.make_async_copy(
                        stage, out_hbm.at[:, pl.ds(n_off, N_HALF)], st_sem)
                    st.start()
                    st.wait()
            return _

        lax.fori_loop(0, 2, run_pass, 0)

    return pl.pallas_call(
        body,
        out_shape=jax.ShapeDtypeStruct((M_PER, N), jnp.float32),
        in_specs=[
            pl.BlockSpec(memory_space=pl.ANY),
            pl.BlockSpec(memory_space=pl.ANY),
        ],
        out_specs=pl.BlockSpec(memory_space=pl.ANY),
        scratch_shapes=[
            pltpu.VMEM((RH, k_per), jnp.bfloat16),
            pltpu.VMEM((RH, k_per), jnp.bfloat16),
            pltpu.VMEM((k_per, N_HALF), jnp.bfloat16),
            pltpu.VMEM((2, RH, N_HALF), jnp.bfloat16),
            pltpu.VMEM((2, RH, N_HALF), jnp.bfloat16),
            pltpu.VMEM((RH, N_HALF), jnp.bfloat16),
            pltpu.VMEM((RH, N_HALF), jnp.bfloat16),
            pltpu.VMEM((M_PER, N_HALF), jnp.float32),
            pltpu.SemaphoreType.DMA,
            pltpu.SemaphoreType.DMA,
            pltpu.SemaphoreType.DMA,
            pltpu.SemaphoreType.DMA,
            pltpu.SemaphoreType.DMA((2,)),
            pltpu.SemaphoreType.DMA((2,)),
            pltpu.SemaphoreType.DMA((2,)),
            pltpu.SemaphoreType.DMA((2,)),
        ],
        compiler_params=pltpu.CompilerParams(
            collective_id=0,
            vmem_limit_bytes=64 * 1024 * 1024,
        ),
    )(x, w)


# device time: 417244 ns/iter; 1.9274x vs baseline; 1.0265x over previous
import jax
import jax.numpy as jnp
from jax import lax
from jax.experimental import pallas as pl
from jax.experimental.pallas import tpu as pltpu

N_DEV = 4
M = 8192
M_PER = M // N_DEV
RH = M_PER // 2
RQ = M_PER // 4
N = 4096
N_HALF = N // 2
RS = 256
DOT_RS = 512


def kernel(x, w_mat):
    x = x.astype(jnp.bfloat16)
    w = w_mat.astype(jnp.bfloat16)
    k_per = x.shape[1]

    def body(x_hbm, w_hbm, out_hbm,
             xa, xb, wbuf, comm_r0, comm_r1, comm_l0, comm_l1,
             pb_r0, pb_r1, pb_l0, pb_l1, outbuf,
             ldw_sem, lda_sem, ldb_sem, out_sems,
             send_r0, recv_r0, send_r1, recv_r1,
             send_l0, recv_l0, send_l1, recv_l1):
        my = lax.axis_index("i")
        left = lax.rem(my + N_DEV - 1, N_DEV)
        right = lax.rem(my + 1, N_DEV)

        barrier = pltpu.get_barrier_semaphore()
        pl.semaphore_signal(barrier, inc=1, device_id=(left,),
                            device_id_type=pl.DeviceIdType.MESH)
        pl.semaphore_signal(barrier, inc=1, device_id=(right,),
                            device_id_type=pl.DeviceIdType.MESH)
        pl.semaphore_wait(barrier, 2)

        def load_half(c, row_off, buf, sem):
            cp = pltpu.make_async_copy(
                x_hbm.at[pl.ds(c * M_PER + row_off, RH), :], buf, sem)
            cp.start()
            return cp

        def dot_into(dst, src, src_off):
            def step(r, carry):
                dst[pl.ds(r * DOT_RS, DOT_RS), :] = jnp.dot(
                    src[pl.ds(src_off + r * DOT_RS, DOT_RS), :], wbuf[:, :],
                    preferred_element_type=jnp.float32).astype(jnp.bfloat16)
                return carry
            lax.fori_loop(0, RQ // DOT_RS, step, 0)

        def acc_mid(comm, slot, pb):
            def step(r, carry):
                sl = pl.ds(r * RS, RS)
                comm[slot, sl, :] = (
                    comm[slot, sl, :].astype(jnp.float32)
                    + pb[sl, :].astype(jnp.float32)).astype(jnp.bfloat16)
                return carry
            lax.fori_loop(0, RQ // RS, step, 0)

        def acc_last(comm, slot, pb, ob_slot):
            def step(r, carry):
                sl = pl.ds(r * RS, RS)
                outbuf[ob_slot, sl, :] = (
                    comm[slot, sl, :].astype(jnp.float32)
                    + pb[sl, :].astype(jnp.float32))
                return carry
            lax.fori_loop(0, RQ // RS, step, 0)

        def out_dma(ob_slot, row_off, n_off):
            return pltpu.make_async_copy(
                outbuf.at[ob_slot],
                out_hbm.at[pl.ds(row_off, RQ), pl.ds(n_off, N_HALF)],
                out_sems.at[ob_slot])

        streams = [
            (comm_r0, pb_r0, send_r0, recv_r0, 0, True),
            (comm_l0, pb_l0, send_l0, recv_l0, 2 * RQ, False),
            (comm_r1, pb_r1, send_r1, recv_r1, RQ, True),
            (comm_l1, pb_l1, send_l1, recv_l1, 3 * RQ, False),
        ]

        def run_pass(p, pcarry):
            n_off = p * N_HALF
            cp = pltpu.make_async_copy(
                w_hbm.at[:, pl.ds(n_off, N_HALF)], wbuf, ldw_sem)
            cp.start()
            cp.wait()

            ca = load_half(lax.rem(my + N_DEV - 1, N_DEV), 0, xa, lda_sem)
            cb = load_half(lax.rem(my + 1, N_DEV), RH, xb, ldb_sem)
            ca.wait()
            dot_into(comm_r0.at[0], xa, 0)
            dot_into(comm_r1.at[0], xa, RQ)
            cb.wait()
            dot_into(comm_l0.at[0], xb, 0)
            dot_into(comm_l1.at[0], xb, RQ)

            for h in range(N_DEV - 1):
                s = h % 2
                d = (h + 1) % 2
                rdmas = []
                for comm, pb, snd, rcv, row_off, rightward in streams:
                    rdma = pltpu.make_async_remote_copy(
                        src_ref=comm.at[s], dst_ref=comm.at[d],
                        send_sem=snd.at[s], recv_sem=rcv.at[d],
                        device_id=(right if rightward else left,),
                        device_id_type=pl.DeviceIdType.MESH)
                    rdma.start()
                    rdmas.append(rdma)

                ca = load_half(lax.rem(my + 2 * N_DEV - 2 - h, N_DEV),
                               0, xa, lda_sem)
                cb = load_half(lax.rem(my + 2 + h, N_DEV), RH, xb, ldb_sem)
                ca.wait()
                dot_into(pb_r0, xa, 0)
                dot_into(pb_r1, xa, RQ)
                cb.wait()
                dot_into(pb_l0, xb, 0)
                dot_into(pb_l1, xb, RQ)

                last = h == N_DEV - 2
                for k, (comm, pb, snd, rcv, row_off, rightward) in enumerate(
                        streams):
                    rdmas[k].wait()
                    if not last:
                        acc_mid(comm, d, pb)
                    else:
                        ob = k % 2
                        if k >= 2:
                            out_dma(ob, 0, n_off).wait()
                        else:
                            @pl.when(p > 0)
                            def _():
                                out_dma(ob, 0, n_off).wait()
                        acc_last(comm, d, pb, ob)
                        dma = out_dma(ob, row_off, n_off)
                        dma.start()
            return pcarry

        lax.fori_loop(0, 2, run_pass, 0)
        out_dma(0, 0, 0).wait()
        out_dma(1, 0, 0).wait()

    return pl.pallas_call(
        body,
        out_shape=jax.ShapeDtypeStruct((M_PER, N), jnp.float32),
        in_specs=[
            pl.BlockSpec(memory_space=pl.ANY),
            pl.BlockSpec(memory_space=pl.ANY),
        ],
        out_specs=pl.BlockSpec(memory_space=pl.ANY),
        scratch_shapes=[
            pltpu.VMEM((RH, k_per), jnp.bfloat16),
            pltpu.VMEM((RH, k_per), jnp.bfloat16),
            pltpu.VMEM((k_per, N_HALF), jnp.bfloat16),
            pltpu.VMEM((2, RQ, N_HALF), jnp.bfloat16),
            pltpu.VMEM((2, RQ, N_HALF), jnp.bfloat16),
            pltpu.VMEM((2, RQ, N_HALF), jnp.bfloat16),
            pltpu.VMEM((2, RQ, N_HALF), jnp.bfloat16),
            pltpu.VMEM((RQ, N_HALF), jnp.bfloat16),
            pltpu.VMEM((RQ, N_HALF), jnp.bfloat16),
            pltpu.VMEM((RQ, N_HALF), jnp.bfloat16),
            pltpu.VMEM((RQ, N_HALF), jnp.bfloat16),
            pltpu.VMEM((2, RQ, N_HALF), jnp.float32),
            pltpu.SemaphoreType.DMA,
            pltpu.SemaphoreType.DMA,
            pltpu.SemaphoreType.DMA,
            pltpu.SemaphoreType.DMA((2,)),
            pltpu.SemaphoreType.DMA((2,)),
            pltpu.SemaphoreType.DMA((2,)),
            pltpu.SemaphoreType.DMA((2,)),
            pltpu.SemaphoreType.DMA((2,)),
            pltpu.SemaphoreType.DMA((2,)),
            pltpu.SemaphoreType.DMA((2,)),
            pltpu.SemaphoreType.DMA((2,)),
            pltpu.SemaphoreType.DMA((2,)),
        ],
        compiler_params=pltpu.CompilerParams(
            collective_id=0,
            vmem_limit_bytes=64 * 1024 * 1024,
        ),
    )(x, w)


# device time: 405025 ns/iter; 1.9856x vs baseline; 1.0302x over previous
import jax
import jax.numpy as jnp
from jax import lax
from jax.experimental import pallas as pl
from jax.experimental.pallas import tpu as pltpu

N_DEV = 4
M = 8192
M_PER = M // N_DEV
RH = M_PER // 2
RQ = M_PER // 4
N = 4096
N_HALF = N // 2
RS = 256
DOT_RS = 512


def kernel(x, w_mat):
    x = x.astype(jnp.bfloat16)
    w = w_mat.astype(jnp.bfloat16)
    k_per = x.shape[1]

    def body(x_hbm, w_hbm, out_hbm,
             xa, xb, wbuf, comm_r0, comm_r1, comm_l0, comm_l1,
             pb_r, pb_l, outbuf,
             ldw_sem, lda_sems, ldb_sems, out_sems,
             send_r0, recv_r0, send_r1, recv_r1,
             send_l0, recv_l0, send_l1, recv_l1):
        my = lax.axis_index("i")
        left = lax.rem(my + N_DEV - 1, N_DEV)
        right = lax.rem(my + 1, N_DEV)

        barrier = pltpu.get_barrier_semaphore()
        pl.semaphore_signal(barrier, inc=1, device_id=(left,),
                            device_id_type=pl.DeviceIdType.MESH)
        pl.semaphore_signal(barrier, inc=1, device_id=(right,),
                            device_id_type=pl.DeviceIdType.MESH)
        pl.semaphore_wait(barrier, 2)

        streams = [
            (comm_r0, send_r0, recv_r0, pb_r, 0, 0, True),
            (comm_l0, send_l0, recv_l0, pb_l, 0, 2 * RQ, False),
            (comm_r1, send_r1, recv_r1, pb_r, RQ, RQ, True),
            (comm_l1, send_l1, recv_l1, pb_l, RQ, 3 * RQ, False),
        ]

        def make_rdma(k, h):
            comm, snd, rcv, _, _, _, rightward = streams[k]
            s = h % 2
            d = (h + 1) % 2
            return pltpu.make_async_remote_copy(
                src_ref=comm.at[s], dst_ref=comm.at[d],
                send_sem=snd.at[s], recv_sem=rcv.at[d],
                device_id=(right if rightward else left,),
                device_id_type=pl.DeviceIdType.MESH)

        def load_half(c, row_off, buf, slot, sems):
            cp = pltpu.make_async_copy(
                x_hbm.at[pl.ds(c * M_PER + row_off, RH), :],
                buf.at[slot], sems.at[slot])
            cp.start()
            return cp

        def dots_into(dst, buf, slot):
            def step(r, carry):
                sl = pl.ds(r * DOT_RS, DOT_RS)
                dst[sl, :] = jnp.dot(
                    buf[slot, sl, :], wbuf[:, :],
                    preferred_element_type=jnp.float32).astype(jnp.bfloat16)
                return carry
            lax.fori_loop(0, RH // DOT_RS, step, 0)

        def acc_mid(comm, slot, pb, pb_off):
            def step(r, carry):
                sl = pl.ds(r * RS, RS)
                comm[slot, sl, :] = (
                    comm[slot, sl, :].astype(jnp.float32)
                    + pb[pl.ds(pb_off + r * RS, RS), :].astype(jnp.float32)
                ).astype(jnp.bfloat16)
                return carry
            lax.fori_loop(0, RQ // RS, step, 0)

        def acc_last(comm, slot, pb, pb_off, ob_slot):
            def step(r, carry):
                sl = pl.ds(r * RS, RS)
                outbuf[ob_slot, sl, :] = (
                    comm[slot, sl, :].astype(jnp.float32)
                    + pb[pl.ds(pb_off + r * RS, RS), :].astype(jnp.float32))
                return carry
            lax.fori_loop(0, RQ // RS, step, 0)

        def out_dma(ob_slot, row_off, n_off):
            return pltpu.make_async_copy(
                outbuf.at[ob_slot],
                out_hbm.at[pl.ds(row_off, RQ), pl.ds(n_off, N_HALF)],
                out_sems.at[ob_slot])

        def run_pass(p, pcarry):
            n_off = p * N_HALF
            cp = pltpu.make_async_copy(
                w_hbm.at[:, pl.ds(n_off, N_HALF)], wbuf, ldw_sem)
            cp.start()
            cp.wait()

            la = {0: load_half(lax.rem(my + N_DEV - 1, N_DEV), 0,
                               xa, 0, lda_sems),
                  1: load_half(lax.rem(my + 2, N_DEV), 0, xa, 1, lda_sems)}
            lb = {0: load_half(lax.rem(my + 1, N_DEV), RH, xb, 0, ldb_sems),
                  1: load_half(lax.rem(my + 2, N_DEV), RH, xb, 1, ldb_sems)}

            la[0].wait()
            dots_into(pb_r, xa, 0)

            def copy_first(comm0, comm1, pb):
                def step(r, carry):
                    sl = pl.ds(r * RS, RS)
                    comm0[0, sl, :] = pb[sl, :]
                    comm1[0, sl, :] = pb[pl.ds(RQ + r * RS, RS), :]
                    return carry
                lax.fori_loop(0, RQ // RS, step, 0)

            copy_first(comm_r0, comm_r1, pb_r)
            cur = {}
            cur[0] = make_rdma(0, 0)
            cur[0].start()
            cur[2] = make_rdma(2, 0)
            cur[2].start()
            lb[0].wait()
            dots_into(pb_l, xb, 0)
            copy_first(comm_l0, comm_l1, pb_l)
            cur[1] = make_rdma(1, 0)
            cur[1].start()
            cur[3] = make_rdma(3, 0)
            cur[3].start()
            la[2] = load_half(lax.rem(my + 1, N_DEV), 0, xa, 0, lda_sems)
            lb[2] = load_half(lax.rem(my + 3, N_DEV), RH, xb, 0, ldb_sems)

            for h in range(N_DEV - 1):
                slot = (h + 1) % 2
                d = (h + 1) % 2
                la[h + 1].wait()
                dots_into(pb_r, xa, slot)
                lb[h + 1].wait()
                dots_into(pb_l, xb, slot)
                if h == 0:
                    la[3] = load_half(my, 0, xa, 1, lda_sems)
                    lb[3] = load_half(my, RH, xb, 1, ldb_sems)

                last = h == N_DEV - 2
                for k, (comm, snd, rcv, pb, pb_off, out_row, _rw) in \
                        enumerate(streams):
                    cur[k].wait()
                    if not last:
                        acc_mid(comm, d, pb, pb_off)
                        cur[k] = make_rdma(k, h + 1)
                        cur[k].start()
                    else:
                        ob = k % 2
                        if k >= 2:
                            out_dma(ob, 0, n_off).wait()
                        else:
                            @pl.when(p > 0)
                            def _():
                                out_dma(ob, 0, n_off).wait()
                        acc_last(comm, d, pb, pb_off, ob)
                        dma = out_dma(ob, out_row, n_off)
                        dma.start()
            return pcarry

        lax.fori_loop(0, 2, run_pass, 0)
        out_dma(0, 0, 0).wait()
        out_dma(1, 0, 0).wait()

    return pl.pallas_call(
        body,
        out_shape=jax.ShapeDtypeStruct((M_PER, N), jnp.float32),
        in_specs=[
            pl.BlockSpec(memory_space=pl.ANY),
            pl.BlockSpec(memory_space=pl.ANY),
        ],
        out_specs=pl.BlockSpec(memory_space=pl.ANY),
        scratch_shapes=[
            pltpu.VMEM((2, RH, k_per), jnp.bfloat16),
            pltpu.VMEM((2, RH, k_per), jnp.bfloat16),
            pltpu.VMEM((k_per, N_HALF), jnp.bfloat16),
            pltpu.VMEM((2, RQ, N_HALF), jnp.bfloat16),
            pltpu.VMEM((2, RQ, N_HALF), jnp.bfloat16),
            pltpu.VMEM((2, RQ, N_HALF), jnp.bfloat16),
            pltpu.VMEM((2, RQ, N_HALF), jnp.bfloat16),
            pltpu.VMEM((RH, N_HALF), jnp.bfloat16),
            pltpu.VMEM((RH, N_HALF), jnp.bfloat16),
            pltpu.VMEM((2, RQ, N_HALF), jnp.float32),
            pltpu.SemaphoreType.DMA,
            pltpu.SemaphoreType.DMA((2,)),
            pltpu.SemaphoreType.DMA((2,)),
            pltpu.SemaphoreType.DMA((2,)),
            pltpu.SemaphoreType.DMA((2,)),
            pltpu.SemaphoreType.DMA((2,)),
            pltpu.SemaphoreType.DMA((2,)),
            pltpu.SemaphoreType.DMA((2,)),
            pltpu.SemaphoreType.DMA((2,)),
            pltpu.SemaphoreType.DMA((2,)),
            pltpu.SemaphoreType.DMA((2,)),
            pltpu.SemaphoreType.DMA((2,)),
        ],
        compiler_params=pltpu.CompilerParams(
            collective_id=0,
            vmem_limit_bytes=64 * 1024 * 1024,
        ),
    )(x, w)


# device time: 374082 ns/iter; 2.1498x vs baseline; 1.0827x over previous
import jax
import jax.numpy as jnp
from jax import lax
from jax.experimental import pallas as pl
from jax.experimental.pallas import tpu as pltpu

N_DEV = 4
M = 8192
M_PER = M // N_DEV
RH = M_PER // 2
RQ = M_PER // 4
N = 4096
N_HALF = N // 2
RS = 256
DOT_RS = 512


def kernel(x, w_mat):
    w = w_mat.astype(jnp.bfloat16)
    k_per = x.shape[1]

    def body(x_hbm, w_hbm, out_hbm,
             xa, xb, wbuf, comm_r0, comm_r1, comm_l0, comm_l1,
             pb_r, pb_l, outbuf,
             ldw_sem, lda_sem, ldb_sem, out_sems,
             send_r0, recv_r0, send_r1, recv_r1,
             send_l0, recv_l0, send_l1, recv_l1):
        my = lax.axis_index("i")
        left = lax.rem(my + N_DEV - 1, N_DEV)
        right = lax.rem(my + 1, N_DEV)

        barrier = pltpu.get_barrier_semaphore()
        pl.semaphore_signal(barrier, inc=1, device_id=(left,),
                            device_id_type=pl.DeviceIdType.MESH)
        pl.semaphore_signal(barrier, inc=1, device_id=(right,),
                            device_id_type=pl.DeviceIdType.MESH)
        pl.semaphore_wait(barrier, 2)

        streams = [
            (comm_r0, send_r0, recv_r0, pb_r, 0, 0, True),
            (comm_l0, send_l0, recv_l0, pb_l, 0, 2 * RQ, False),
            (comm_r1, send_r1, recv_r1, pb_r, RQ, RQ, True),
            (comm_l1, send_l1, recv_l1, pb_l, RQ, 3 * RQ, False),
        ]

        def make_rdma(k, h):
            comm, snd, rcv, _, _, _, rightward = streams[k]
            s = h % 2
            d = (h + 1) % 2
            return pltpu.make_async_remote_copy(
                src_ref=comm.at[s], dst_ref=comm.at[d],
                send_sem=snd.at[s], recv_sem=rcv.at[d],
                device_id=(right if rightward else left,),
                device_id_type=pl.DeviceIdType.MESH)

        def load_half(c, row_off, buf, sem):
            cp = pltpu.make_async_copy(
                x_hbm.at[pl.ds(c * M_PER + row_off, RH), :], buf, sem)
            cp.start()
            return cp

        def dots_into(dst, buf):
            def step(r, carry):
                sl = pl.ds(r * DOT_RS, DOT_RS)
                dst[sl, :] = jnp.dot(
                    buf[sl, :].astype(jnp.bfloat16), wbuf[:, :],
                    preferred_element_type=jnp.float32).astype(jnp.bfloat16)
                return carry
            lax.fori_loop(0, RH // DOT_RS, step, 0)

        def acc_mid(comm, slot, pb, pb_off):
            def step(r, carry):
                sl = pl.ds(r * RS, RS)
                comm[slot, sl, :] = (
                    comm[slot, sl, :].astype(jnp.float32)
                    + pb[pl.ds(pb_off + r * RS, RS), :].astype(jnp.float32)
                ).astype(jnp.bfloat16)
                return carry
            lax.fori_loop(0, RQ // RS, step, 0)

        def acc_last(comm, slot, pb, pb_off, ob_slot):
            def step(r, carry):
                sl = pl.ds(r * RS, RS)
                outbuf[ob_slot, sl, :] = (
                    comm[slot, sl, :].astype(jnp.float32)
                    + pb[pl.ds(pb_off + r * RS, RS), :].astype(jnp.float32))
                return carry
            lax.fori_loop(0, RQ // RS, step, 0)

        def out_dma(ob_slot, row_off, n_off):
            return pltpu.make_async_copy(
                outbuf.at[ob_slot],
                out_hbm.at[pl.ds(row_off, RQ), pl.ds(n_off, N_HALF)],
                out_sems.at[ob_slot])

        def run_pass(p, pcarry):
            n_off = p * N_HALF
            cp = pltpu.make_async_copy(
                w_hbm.at[:, pl.ds(n_off, N_HALF)], wbuf, ldw_sem)
            cp.start()
            cp.wait()

            la = load_half(lax.rem(my + N_DEV - 1, N_DEV), 0, xa, lda_sem)
            lb = load_half(lax.rem(my + 1, N_DEV), RH, xb, ldb_sem)

            def copy_first(comm0, comm1, pb):
                def step(r, carry):
                    sl = pl.ds(r * RS, RS)
                    comm0[0, sl, :] = pb[sl, :]
                    comm1[0, sl, :] = pb[pl.ds(RQ + r * RS, RS), :]
                    return carry
                lax.fori_loop(0, RQ // RS, step, 0)

            cur = {}
            la.wait()
            dots_into(pb_r, xa)
            la = load_half(lax.rem(my + 2, N_DEV), 0, xa, lda_sem)
            copy_first(comm_r0, comm_r1, pb_r)
            cur[0] = make_rdma(0, 0)
            cur[0].start()
            cur[2] = make_rdma(2, 0)
            cur[2].start()
            lb.wait()
            dots_into(pb_l, xb)
            lb = load_half(lax.rem(my + 2, N_DEV), RH, xb, ldb_sem)
            copy_first(comm_l0, comm_l1, pb_l)
            cur[1] = make_rdma(1, 0)
            cur[1].start()
            cur[3] = make_rdma(3, 0)
            cur[3].start()

            for h in range(N_DEV - 1):
                d = (h + 1) % 2
                la.wait()
                dots_into(pb_r, xa)
                lb.wait()
                dots_into(pb_l, xb)
                if h < N_DEV - 2:
                    la = load_half(lax.rem(my + N_DEV + 1 - h, N_DEV),
                                   0, xa, lda_sem)
                    lb = load_half(lax.rem(my + 3 + h, N_DEV),
                                   RH, xb, ldb_sem)

                last = h == N_DEV - 2
                for k, (comm, snd, rcv, pb, pb_off, out_row, _rw) in \
                        enumerate(streams):
                    cur[k].wait()
                    if not last:
                        acc_mid(comm, d, pb, pb_off)
                        cur[k] = make_rdma(k, h + 1)
                        cur[k].start()
                    else:
                        ob = k % 2
                        if k >= 2:
                            out_dma(ob, 0, n_off).wait()
                        else:
                            @pl.when(p > 0)
                            def _():
                                out_dma(ob, 0, n_off).wait()
                        acc_last(comm, d, pb, pb_off, ob)
                        dma = out_dma(ob, out_row, n_off)
                        dma.start()
            return pcarry

        lax.fori_loop(0, 2, run_pass, 0)
        out_dma(0, 0, 0).wait()
        out_dma(1, 0, 0).wait()

    return pl.pallas_call(
        body,
        out_shape=jax.ShapeDtypeStruct((M_PER, N), jnp.float32),
        in_specs=[
            pl.BlockSpec(memory_space=pl.ANY),
            pl.BlockSpec(memory_space=pl.ANY),
        ],
        out_specs=pl.BlockSpec(memory_space=pl.ANY),
        scratch_shapes=[
            pltpu.VMEM((RH, k_per), jnp.float32),
            pltpu.VMEM((RH, k_per), jnp.float32),
            pltpu.VMEM((k_per, N_HALF), jnp.bfloat16),
            pltpu.VMEM((2, RQ, N_HALF), jnp.bfloat16),
            pltpu.VMEM((2, RQ, N_HALF), jnp.bfloat16),
            pltpu.VMEM((2, RQ, N_HALF), jnp.bfloat16),
            pltpu.VMEM((2, RQ, N_HALF), jnp.bfloat16),
            pltpu.VMEM((RH, N_HALF), jnp.bfloat16),
            pltpu.VMEM((RH, N_HALF), jnp.bfloat16),
            pltpu.VMEM((2, RQ, N_HALF), jnp.float32),
            pltpu.SemaphoreType.DMA,
            pltpu.SemaphoreType.DMA,
            pltpu.SemaphoreType.DMA,
            pltpu.SemaphoreType.DMA((2,)),
            pltpu.SemaphoreType.DMA((2,)),
            pltpu.SemaphoreType.DMA((2,)),
            pltpu.SemaphoreType.DMA((2,)),
            pltpu.SemaphoreType.DMA((2,)),
            pltpu.SemaphoreType.DMA((2,)),
            pltpu.SemaphoreType.DMA((2,)),
            pltpu.SemaphoreType.DMA((2,)),
            pltpu.SemaphoreType.DMA((2,)),
        ],
        compiler_params=pltpu.CompilerParams(
            collective_id=0,
            vmem_limit_bytes=64 * 1024 * 1024,
        ),
    )(x, w)


# device time: 354118 ns/iter; 2.2710x vs baseline; 1.0564x over previous
import jax
import jax.numpy as jnp
from jax import lax
from jax.experimental import pallas as pl
from jax.experimental.pallas import tpu as pltpu

N_DEV = 4
M = 8192
M_PER = M // N_DEV
RH = M_PER // 2
RQ = M_PER // 4
N = 4096
N_HALF = N // 2
RS = 256
DOT_RS = 512


def kernel(x, w_mat):
    w = w_mat.astype(jnp.bfloat16)
    k_per = x.shape[1]

    def body(x_hbm, w_hbm, out_hbm,
             xa, xb, wbuf, comm_r0, comm_r1, comm_l0, comm_l1,
             pb_r, pb_l, outbuf,
             ldw_sem, lda_sem, ldb_sem, out_sems,
             send_r0, recv_r0, send_r1, recv_r1,
             send_l0, recv_l0, send_l1, recv_l1):
        my = lax.axis_index("i")
        left = lax.rem(my + N_DEV - 1, N_DEV)
        right = lax.rem(my + 1, N_DEV)

        barrier = pltpu.get_barrier_semaphore()
        pl.semaphore_signal(barrier, inc=1, device_id=(left,),
                            device_id_type=pl.DeviceIdType.MESH)
        pl.semaphore_signal(barrier, inc=1, device_id=(right,),
                            device_id_type=pl.DeviceIdType.MESH)
        pl.semaphore_wait(barrier, 2)

        streams = [
            (comm_r0, send_r0, recv_r0, pb_r, 0, 0, True),
            (comm_l0, send_l0, recv_l0, pb_l, 0, 2 * RQ, False),
            (comm_r1, send_r1, recv_r1, pb_r, RQ, RQ, True),
            (comm_l1, send_l1, recv_l1, pb_l, RQ, 3 * RQ, False),
        ]

        def make_rdma(k, h):
            comm, snd, rcv, _, _, _, rightward = streams[k]
            s = h % 2
            d = (h + 1) % 2
            return pltpu.make_async_remote_copy(
                src_ref=comm.at[s], dst_ref=comm.at[d],
                send_sem=snd.at[s], recv_sem=rcv.at[d],
                device_id=(right if rightward else left,),
                device_id_type=pl.DeviceIdType.MESH)

        def load_half(c, row_off, buf, sem):
            cp = pltpu.make_async_copy(
                x_hbm.at[pl.ds(c * M_PER + row_off, RH), :], buf, sem)
            cp.start()
            return cp

        def dots_into(dst, buf):
            def step(r, carry):
                sl = pl.ds(r * DOT_RS, DOT_RS)
                dst[sl, :] = jnp.dot(
                    buf[sl, :].astype(jnp.bfloat16), wbuf[:, :],
                    preferred_element_type=jnp.float32).astype(jnp.bfloat16)
                return carry
            lax.fori_loop(0, RH // DOT_RS, step, 0)

        def acc_mid(comm, slot, pb, pb_off):
            def step(r, carry):
                sl = pl.ds(r * RS, RS)
                comm[slot, sl, :] = (
                    comm[slot, sl, :].astype(jnp.float32)
                    + pb[pl.ds(pb_off + r * RS, RS), :].astype(jnp.float32)
                ).astype(jnp.bfloat16)
                return carry
            lax.fori_loop(0, RQ // RS, step, 0)

        def acc_last(comm, slot, pb, pb_off, ob_slot):
            def step(r, carry):
                sl = pl.ds(r * RS, RS)
                outbuf[ob_slot, sl, :] = (
                    comm[slot, sl, :].astype(jnp.float32)
                    + pb[pl.ds(pb_off + r * RS, RS), :].astype(jnp.float32))
                return carry
            lax.fori_loop(0, RQ // RS, step, 0)

        def out_dma(ob_slot, row_off, n_off):
            return pltpu.make_async_copy(
                outbuf.at[ob_slot],
                out_hbm.at[pl.ds(row_off, RQ), pl.ds(n_off, N_HALF)],
                out_sems.at[ob_slot])

        def dot_q(dst, buf, row_off):
            dst[:, :] = jnp.dot(
                buf[pl.ds(row_off, RQ), :].astype(jnp.bfloat16), wbuf[:, :],
                preferred_element_type=jnp.float32).astype(jnp.bfloat16)

        def load_w(p):
            cp = pltpu.make_async_copy(
                w_hbm.at[:, pl.ds(p * N_HALF, N_HALF)], wbuf, ldw_sem)
            cp.start()
            return cp

        wcp = load_w(0)
        la = load_half(lax.rem(my + N_DEV - 1, N_DEV), 0, xa, lda_sem)
        lb = load_half(lax.rem(my + 1, N_DEV), RH, xb, ldb_sem)

        for p in range(2):
            n_off = p * N_HALF
            cur = {}
            wcp.wait()
            la.wait()
            dot_q(comm_r0.at[0], xa, 0)
            cur[0] = make_rdma(0, 0)
            cur[0].start()
            dot_q(comm_r1.at[0], xa, RQ)
            cur[2] = make_rdma(2, 0)
            cur[2].start()
            la = load_half(lax.rem(my + 2, N_DEV), 0, xa, lda_sem)
            lb.wait()
            dot_q(comm_l0.at[0], xb, 0)
            cur[1] = make_rdma(1, 0)
            cur[1].start()
            dot_q(comm_l1.at[0], xb, RQ)
            cur[3] = make_rdma(3, 0)
            cur[3].start()
            lb = load_half(lax.rem(my + 2, N_DEV), RH, xb, ldb_sem)

            for h in range(N_DEV - 1):
                d = (h + 1) % 2
                la.wait()
                dots_into(pb_r, xa)
                lb.wait()
                dots_into(pb_l, xb)
                if h < N_DEV - 2:
                    la = load_half(lax.rem(my + N_DEV + 1 - h, N_DEV),
                                   0, xa, lda_sem)
                    lb = load_half(lax.rem(my + 3 + h, N_DEV),
                                   RH, xb, ldb_sem)
                elif p == 0:
                    wcp = load_w(1)
                    la = load_half(lax.rem(my + N_DEV - 1, N_DEV),
                                   0, xa, lda_sem)
                    lb = load_half(lax.rem(my + 1, N_DEV), RH, xb, ldb_sem)

                last = h == N_DEV - 2
                for k, (comm, snd, rcv, pb, pb_off, out_row, _rw) in \
                        enumerate(streams):
                    cur[k].wait()
                    if not last:
                        acc_mid(comm, d, pb, pb_off)
                        cur[k] = make_rdma(k, h + 1)
                        cur[k].start()
                    else:
                        ob = k % 2
                        if k >= 2 or p > 0:
                            out_dma(ob, 0, n_off).wait()
                        acc_last(comm, d, pb, pb_off, ob)
                        dma = out_dma(ob, out_row, n_off)
                        dma.start()
        out_dma(0, 0, 0).wait()
        out_dma(1, 0, 0).wait()

    return pl.pallas_call(
        body,
        out_shape=jax.ShapeDtypeStruct((M_PER, N), jnp.float32),
        in_specs=[
            pl.BlockSpec(memory_space=pl.ANY),
            pl.BlockSpec(memory_space=pl.ANY),
        ],
        out_specs=pl.BlockSpec(memory_space=pl.ANY),
        scratch_shapes=[
            pltpu.VMEM((RH, k_per), jnp.float32),
            pltpu.VMEM((RH, k_per), jnp.float32),
            pltpu.VMEM((k_per, N_HALF), jnp.bfloat16),
            pltpu.VMEM((2, RQ, N_HALF), jnp.bfloat16),
            pltpu.VMEM((2, RQ, N_HALF), jnp.bfloat16),
            pltpu.VMEM((2, RQ, N_HALF), jnp.bfloat16),
            pltpu.VMEM((2, RQ, N_HALF), jnp.bfloat16),
            pltpu.VMEM((RH, N_HALF), jnp.bfloat16),
            pltpu.VMEM((RH, N_HALF), jnp.bfloat16),
            pltpu.VMEM((2, RQ, N_HALF), jnp.float32),
            pltpu.SemaphoreType.DMA,
            pltpu.SemaphoreType.DMA,
            pltpu.SemaphoreType.DMA,
            pltpu.SemaphoreType.DMA((2,)),
            pltpu.SemaphoreType.DMA((2,)),
            pltpu.SemaphoreType.DMA((2,)),
            pltpu.SemaphoreType.DMA((2,)),
            pltpu.SemaphoreType.DMA((2,)),
            pltpu.SemaphoreType.DMA((2,)),
            pltpu.SemaphoreType.DMA((2,)),
            pltpu.SemaphoreType.DMA((2,)),
            pltpu.SemaphoreType.DMA((2,)),
        ],
        compiler_params=pltpu.CompilerParams(
            collective_id=0,
            vmem_limit_bytes=64 * 1024 * 1024,
        ),
    )(x, w)


# device time: 353728 ns/iter; 2.2735x vs baseline; 1.0011x over previous
import jax
import jax.numpy as jnp
from jax import lax
from jax.experimental import pallas as pl
from jax.experimental.pallas import tpu as pltpu

N_DEV = 4
M = 8192
M_PER = M // N_DEV
RH = M_PER // 2
RQ = M_PER // 4
N = 4096
N_HALF = N // 2
RS = 256
DOT_RS = 512


def kernel(x, w_mat):
    w = w_mat.astype(jnp.bfloat16)
    k_per = x.shape[1]

    def body(x_hbm, w_hbm, out_hbm,
             xa, xb, wbuf, comm_r0, comm_r1, comm_l0, comm_l1,
             pb_r, pb_l, outbuf,
             ldw_sem, lda_sem, ldb_sem, out_sems,
             send_r0, recv_r0, send_r1, recv_r1,
             send_l0, recv_l0, send_l1, recv_l1):
        my = lax.axis_index("i")
        left = lax.rem(my + N_DEV - 1, N_DEV)
        right = lax.rem(my + 1, N_DEV)

        barrier = pltpu.get_barrier_semaphore()
        pl.semaphore_signal(barrier, inc=1, device_id=(left,),
                            device_id_type=pl.DeviceIdType.MESH)
        pl.semaphore_signal(barrier, inc=1, device_id=(right,),
                            device_id_type=pl.DeviceIdType.MESH)

        streams = [
            (comm_r0, send_r0, recv_r0, pb_r, 0, 0, True),
            (comm_l0, send_l0, recv_l0, pb_l, 0, 2 * RQ, False),
            (comm_r1, send_r1, recv_r1, pb_r, RQ, RQ, True),
            (comm_l1, send_l1, recv_l1, pb_l, RQ, 3 * RQ, False),
        ]

        def make_rdma(k, h):
            comm, snd, rcv, _, _, _, rightward = streams[k]
            s = h % 2
            d = (h + 1) % 2
            return pltpu.make_async_remote_copy(
                src_ref=comm.at[s], dst_ref=comm.at[d],
                send_sem=snd.at[s], recv_sem=rcv.at[d],
                device_id=(right if rightward else left,),
                device_id_type=pl.DeviceIdType.MESH)

        def load_half(c, row_off, buf, sem):
            cp = pltpu.make_async_copy(
                x_hbm.at[pl.ds(c * M_PER + row_off, RH), :], buf, sem)
            cp.start()
            return cp

        def dots_into(dst, buf):
            def step(r, carry):
                sl = pl.ds(r * DOT_RS, DOT_RS)
                dst[sl, :] = jnp.dot(
                    buf[sl, :].astype(jnp.bfloat16), wbuf[:, :],
                    preferred_element_type=jnp.float32).astype(jnp.bfloat16)
                return carry
            lax.fori_loop(0, RH // DOT_RS, step, 0)

        def acc_mid(comm, slot, pb, pb_off):
            def step(r, carry):
                sl = pl.ds(r * RS, RS)
                comm[slot, sl, :] = (
                    comm[slot, sl, :].astype(jnp.float32)
                    + pb[pl.ds(pb_off + r * RS, RS), :].astype(jnp.float32)
                ).astype(jnp.bfloat16)
                return carry
            lax.fori_loop(0, RQ // RS, step, 0)

        def acc_last(comm, slot, pb, pb_off, ob_slot):
            def step(r, carry):
                sl = pl.ds(r * RS, RS)
                outbuf[ob_slot, sl, :] = (
                    comm[slot, sl, :].astype(jnp.float32)
                    + pb[pl.ds(pb_off + r * RS, RS), :].astype(jnp.float32))
                return carry
            lax.fori_loop(0, RQ // RS, step, 0)

        def out_dma(ob_slot, row_off, n_off):
            return pltpu.make_async_copy(
                outbuf.at[ob_slot],
                out_hbm.at[pl.ds(row_off, RQ), pl.ds(n_off, N_HALF)],
                out_sems.at[ob_slot])

        def dot_q(dst, buf, row_off):
            dst[:, :] = jnp.dot(
                buf[pl.ds(row_off, RQ), :].astype(jnp.bfloat16), wbuf[:, :],
                preferred_element_type=jnp.float32).astype(jnp.bfloat16)

        def load_w(p):
            cp = pltpu.make_async_copy(
                w_hbm.at[:, pl.ds(p * N_HALF, N_HALF)], wbuf, ldw_sem)
            cp.start()
            return cp

        wcp = load_w(0)
        la = load_half(lax.rem(my + N_DEV - 1, N_DEV), 0, xa, lda_sem)
        lb = load_half(lax.rem(my + 1, N_DEV), RH, xb, ldb_sem)

        for p in range(2):
            n_off = p * N_HALF
            cur = {}
            wcp.wait()
            la.wait()
            dot_q(comm_r0.at[0], xa, 0)
            if p == 0:
                pl.semaphore_wait(barrier, 2)
            cur[0] = make_rdma(0, 0)
            cur[0].start()
            dot_q(comm_r1.at[0], xa, RQ)
            cur[2] = make_rdma(2, 0)
            cur[2].start()
            la = load_half(lax.rem(my + 2, N_DEV), 0, xa, lda_sem)
            lb.wait()
            dot_q(comm_l0.at[0], xb, 0)
            cur[1] = make_rdma(1, 0)
            cur[1].start()
            dot_q(comm_l1.at[0], xb, RQ)
            cur[3] = make_rdma(3, 0)
            cur[3].start()
            lb = load_half(lax.rem(my + 2, N_DEV), RH, xb, ldb_sem)

            for h in range(N_DEV - 1):
                d = (h + 1) % 2
                la.wait()
                dots_into(pb_r, xa)
                lb.wait()
                dots_into(pb_l, xb)
                if h < N_DEV - 2:
                    la = load_half(lax.rem(my + N_DEV + 1 - h, N_DEV),
                                   0, xa, lda_sem)
                    lb = load_half(lax.rem(my + 3 + h, N_DEV),
                                   RH, xb, ldb_sem)
                elif p == 0:
                    wcp = load_w(1)
                    la = load_half(lax.rem(my + N_DEV - 1, N_DEV),
                                   0, xa, lda_sem)
                    lb = load_half(lax.rem(my + 1, N_DEV), RH, xb, ldb_sem)

                last = h == N_DEV - 2
                for k, (comm, snd, rcv, pb, pb_off, out_row, _rw) in \
                        enumerate(streams):
                    cur[k].wait()
                    if not last:
                        acc_mid(comm, d, pb, pb_off)
                        cur[k] = make_rdma(k, h + 1)
                        cur[k].start()
                    else:
                        ob = k % 2
                        if k >= 2 or p > 0:
                            out_dma(ob, 0, n_off).wait()
                        acc_last(comm, d, pb, pb_off, ob)
                        dma = out_dma(ob, out_row, n_off)
                        dma.start()
        out_dma(0, 0, 0).wait()
        out_dma(1, 0, 0).wait()

    return pl.pallas_call(
        body,
        out_shape=jax.ShapeDtypeStruct((M_PER, N), jnp.float32),
        in_specs=[
            pl.BlockSpec(memory_space=pl.ANY),
            pl.BlockSpec(memory_space=pl.ANY),
        ],
        out_specs=pl.BlockSpec(memory_space=pl.ANY),
        scratch_shapes=[
            pltpu.VMEM((RH, k_per), jnp.float32),
            pltpu.VMEM((RH, k_per), jnp.float32),
            pltpu.VMEM((k_per, N_HALF), jnp.bfloat16),
            pltpu.VMEM((2, RQ, N_HALF), jnp.bfloat16),
            pltpu.VMEM((2, RQ, N_HALF), jnp.bfloat16),
            pltpu.VMEM((2, RQ, N_HALF), jnp.bfloat16),
            pltpu.VMEM((2, RQ, N_HALF), jnp.bfloat16),
            pltpu.VMEM((RH, N_HALF), jnp.bfloat16),
            pltpu.VMEM((RH, N_HALF), jnp.bfloat16),
            pltpu.VMEM((2, RQ, N_HALF), jnp.float32),
            pltpu.SemaphoreType.DMA,
            pltpu.SemaphoreType.DMA,
            pltpu.SemaphoreType.DMA,
            pltpu.SemaphoreType.DMA((2,)),
            pltpu.SemaphoreType.DMA((2,)),
            pltpu.SemaphoreType.DMA((2,)),
            pltpu.SemaphoreType.DMA((2,)),
            pltpu.SemaphoreType.DMA((2,)),
            pltpu.SemaphoreType.DMA((2,)),
            pltpu.SemaphoreType.DMA((2,)),
            pltpu.SemaphoreType.DMA((2,)),
            pltpu.SemaphoreType.DMA((2,)),
        ],
        compiler_params=pltpu.CompilerParams(
            collective_id=0,
            vmem_limit_bytes=64 * 1024 * 1024,
        ),
    )(x, w)
